# baseline (speedup 1.0000x reference)
import functools

import jax
import jax.numpy as jnp
from jax import lax
from jax.experimental import pallas as pl
from jax.experimental.pallas import tpu as pltpu
from jax.experimental.pallas import tpu_sc as plsc

TOK_DIM = 57
STRUCT_DIM = 7
D_MODEL = 64

_info = plsc.get_sparse_core_info()
NC, NS, NLANES = _info.num_cores, _info.num_subcores, _info.num_lanes
NW = NC * NS

CHUNK = 128


def _sc_embed(BL):
    per_w = BL // NW
    n_chunks = per_w // CHUNK
    mesh = plsc.VectorSubcoreMesh(core_axis_name="c", subcore_axis_name="s")

    @functools.partial(
        pl.kernel,
        mesh=mesh,
        compiler_params=pltpu.CompilerParams(needs_layout_passes=False),
        out_type=jax.ShapeDtypeStruct((BL // 2, 2 * D_MODEL), jnp.float32),
        scratch_types=[
            pltpu.VMEM((CHUNK,), jnp.int32),
            pltpu.VMEM((CHUNK, 2 * D_MODEL), jnp.float32),
            pltpu.VMEM((CHUNK // 2, 2 * D_MODEL), jnp.float32),
            pltpu.VMEM((CHUNK * STRUCT_DIM,), jnp.float32),
            pltpu.VMEM((56 * NLANES,), jnp.float32),
            pltpu.SemaphoreType.DMA,
        ],
    )
    def k(tok_hbm, x_hbm, tab_hbm, wb_hbm, out_hbm,
          tokv, rows, outv, xv, wbv, sem):
        wid = lax.axis_index("s") * NC + lax.axis_index("c")
        wbase = wid * per_w
        pltpu.sync_copy(wb_hbm, wbv)

        def body(c, _):
            base = pl.multiple_of(wbase + c * CHUNK, CHUNK)
            pltpu.sync_copy(tok_hbm.at[pl.ds(base, CHUNK)], tokv)
            pltpu.async_copy(tab_hbm.at[tokv], rows, sem).wait()
            pltpu.sync_copy(
                x_hbm.at[pl.ds(base * STRUCT_DIM, CHUNK * STRUCT_DIM)], xv)
            for r in range(CHUNK):
                i, h = r // 2, (r % 2) * D_MODEL
                for q in range(D_MODEL // NLANES):
                    outv[i, pl.ds(h + NLANES * q, NLANES)] = (
                        rows[r, pl.ds(NLANES * q, NLANES)])
            for g in range(CHUNK // NLANES):
                ridx = g * NLANES + lax.iota(jnp.int32, NLANES)
                fidx = ridx * STRUCT_DIM
                xd = [plsc.load_gather(xv, [fidx + d])
                      for d in range(STRUCT_DIM)]
                orow = ridx // 2
                ocol = (ridx % 2) * D_MODEL + TOK_DIM
                for e in range(STRUCT_DIM):
                    acc = wbv[pl.ds((49 + e) * NLANES, NLANES)]
                    for d in range(STRUCT_DIM):
                        acc = acc + xd[d] * wbv[
                            pl.ds((e * STRUCT_DIM + d) * NLANES, NLANES)]
                    plsc.store_scatter(outv, [orow, ocol + e], acc)
            pltpu.sync_copy(
                outv,
                out_hbm.at[pl.ds(pl.multiple_of(base // 2, CHUNK // 2),
                                 CHUNK // 2)])
            return ()

        lax.fori_loop(0, n_chunks, body, ())

    return k


def kernel(tokens, structure_features, table, W, b):
    B, L = tokens.shape
    BL = B * L
    tok_flat = tokens.reshape(BL).astype(jnp.int32)
    x_flat = structure_features.reshape(BL * STRUCT_DIM)
    wb = jnp.broadcast_to(
        jnp.concatenate([W.reshape(-1), b], axis=0)[:, None],
        (56, NLANES)).reshape(-1)
    tab64 = jnp.pad(table, ((0, 0), (0, D_MODEL - TOK_DIM)))
    tab2 = jnp.concatenate([tab64, jnp.roll(tab64, -1, axis=0)], axis=1)
    out2 = _sc_embed(BL)(tok_flat, x_flat, tab2, wb)
    return out2.reshape(B, L, D_MODEL)
